# baseline (speedup 1.0000x reference)
import functools

import jax
import jax.numpy as jnp
from jax import lax
from jax.experimental import pallas as pl
from jax.experimental.pallas import tpu as pltpu
from jax.experimental.pallas import tpu_sc as plsc

VOCAB = 1_000_000
D = 64
BATCH = 4096
SEQ = 200

NC = 2
NS = 16
NW = NC * NS
B_PER_W = BATCH // NW
G1 = 128
G2 = SEQ - G1
NBUF = 3


_TBLK = 1024


def _tc_transpose_body(tt_ref, out_ref):
    t = jnp.swapaxes(tt_ref[...], 0, 1)
    out_ref[...] = jnp.concatenate([t, t], axis=1)


_tc_transpose = pl.pallas_call(
    _tc_transpose_body,
    out_shape=jax.ShapeDtypeStruct((VOCAB, 2 * D), jnp.float32),
    grid=(pl.cdiv(VOCAB, _TBLK),),
    in_specs=[pl.BlockSpec((D, _TBLK), lambda i: (0, i))],
    out_specs=pl.BlockSpec((_TBLK, 2 * D), lambda i: (i, 0)),
)


_mesh = plsc.VectorSubcoreMesh(
    core_axis_name="c", subcore_axis_name="s", num_cores=NC, num_subcores=NS
)


@functools.partial(
    pl.kernel,
    out_type=jax.ShapeDtypeStruct((BATCH, SEQ, D), jnp.float32),
    mesh=_mesh,
    scratch_types=[
        pltpu.VMEM((B_PER_W, SEQ), jnp.int32),
        pltpu.VMEM((NBUF, SEQ, D), jnp.float32),
        pltpu.SemaphoreType.DMA,
        pltpu.SemaphoreType.DMA,
        pltpu.SemaphoreType.DMA,
        pltpu.SemaphoreType.DMA,
        pltpu.SemaphoreType.DMA,
        pltpu.SemaphoreType.DMA,
    ],
    compiler_params=pltpu.CompilerParams(use_tc_tiling_on_sc=False),
)
def _sc_gather(table_hbm, idx_hbm, out_hbm, idx_v, rows_v, g0, g1, g2, s0, s1, s2):
    gat_sems = (g0, g1, g2)
    st_sems = (s0, s1, s2)
    wid = lax.axis_index("s") * NC + lax.axis_index("c")
    base_b = wid * B_PER_W

    pltpu.sync_copy(idx_hbm.at[pl.ds(base_b, B_PER_W)], idx_v)

    def issue_gathers(i, b):
        pltpu.async_copy(
            table_hbm.at[idx_v.at[i, pl.ds(0, G1)]],
            rows_v.at[b, pl.ds(0, G1)],
            gat_sems[b],
        )
        pltpu.async_copy(
            table_hbm.at[idx_v.at[i, pl.ds(G1, G2)]],
            rows_v.at[b, pl.ds(G1, G2)],
            gat_sems[b],
        )

    def wait_gathers(b):
        pltpu.make_async_copy(
            table_hbm.at[pl.ds(0, G1)], rows_v.at[b, pl.ds(0, G1)], gat_sems[b]
        ).wait()
        pltpu.make_async_copy(
            table_hbm.at[pl.ds(0, G2)], rows_v.at[b, pl.ds(G1, G2)], gat_sems[b]
        ).wait()

    def issue_store(i, b):
        pltpu.async_copy(rows_v.at[b], out_hbm.at[base_b + i], st_sems[b])

    def wait_store(b):
        pltpu.make_async_copy(rows_v.at[b], out_hbm.at[0], st_sems[b]).wait()

    issue_gathers(0, 0)
    issue_gathers(1, 1)

    wait_gathers(0)
    issue_store(0, 0)
    issue_gathers(2, 2)

    def slot(i, bufs):
        b, bp = bufs
        wait_gathers(b)
        issue_store(i, b)
        wait_store(bp)
        issue_gathers(i + 2, bp)

    def body3(k, carry):
        i = 1 + 3 * k
        slot(i, (1, 0))
        slot(i + 1, (2, 1))
        slot(i + 2, (0, 2))
        return carry

    n3 = (B_PER_W - 3) // 3
    lax.fori_loop(0, n3, body3, 0)
    for i in range(1 + 3 * n3, B_PER_W - 2):
        slot(i, (i % 3, (i - 1) % 3))

    for i in range(B_PER_W - 2, B_PER_W):
        b = i % 3
        wait_gathers(b)
        issue_store(i, b)
        wait_store((i - 1) % 3)
    wait_store((B_PER_W - 1) % 3)


def kernel(x, table):
    table_p = _tc_transpose(table.T).reshape(2 * VOCAB, D)
    x2 = x * 2
    return _sc_gather(table_p, x2)
